# baseline (device time: 609471 ns/iter reference)
import jax
import jax.numpy as jnp
from jax import lax
from jax.experimental import pallas as pl
from jax.experimental.pallas import tpu as pltpu

N_DEV = 4
T_CORR = 64


def kernel(x, A, B, C):
    b, s, d = x.shape
    n = A.shape[1]

    cdt = jnp.bfloat16
    Bt = jnp.swapaxes(B, 1, 2).astype(cdt)
    Cr = C.astype(cdt)
    dA = jnp.exp(A.T).astype(cdt)

    def body(x_ref, dA_ref, B_ref, C_ref, out_ref,
             hout_ref, comm_ref, send_sem, recv_sem):
        my = lax.axis_index("i")
        left = (my - 1) % N_DEV
        right = (my + 1) % N_DEV

        barrier_sem = pltpu.get_barrier_semaphore()
        pl.semaphore_signal(barrier_sem, inc=1, device_id=(left,),
                            device_id_type=pl.DeviceIdType.MESH)
        pl.semaphore_signal(barrier_sem, inc=1, device_id=(right,),
                            device_id_type=pl.DeviceIdType.MESH)
        pl.semaphore_wait(barrier_sem, 2)

        dAv = dA_ref[:, :]

        BLK = 128

        def block_step(k, hs):
            t0 = k * BLK
            bblk = B_ref[:, :, pl.ds(t0, BLK)]
            for j8 in range(0, BLK, 8):
                xchunk = x_ref[:, pl.ds(t0 + j8, 8), :]
                cchunk = C_ref[:, pl.ds(t0 + j8, 8), :]
                ys = [[] for _ in range(b)]
                for jj in range(8):
                    j = j8 + jj
                    new_hs = []
                    for bi in range(b):
                        bt = bblk[bi, :, j:j + 1]
                        xt = xchunk[bi, jj:jj + 1, :]
                        ct = cchunk[bi, jj:jj + 1, :]
                        u = jax.lax.dot_general(
                            bt, xt, (((1,), (0,)), ((), ())),
                            preferred_element_type=jnp.float32
                        ).astype(cdt)
                        h = hs[bi] * dAv + u
                        y = jax.lax.dot_general(
                            ct, h, (((1,), (0,)), ((), ())),
                            preferred_element_type=jnp.float32
                        ).astype(cdt)
                        ys[bi].append(y)
                        new_hs.append(h)
                    hs = tuple(new_hs)
                for bi in range(b):
                    out_ref[bi, pl.ds(t0 + j8, 8), :] = jnp.concatenate(
                        ys[bi], axis=0)
            return hs

        h0 = tuple(jnp.zeros((n, d), dtype=cdt) for _ in range(b))
        h_final = lax.fori_loop(0, s // BLK, block_step, h0)
        for bi in range(b):
            hout_ref[bi] = h_final[bi]

        rdma = pltpu.make_async_remote_copy(
            src_ref=hout_ref,
            dst_ref=comm_ref,
            send_sem=send_sem,
            recv_sem=recv_sem,
            device_id=(right,),
            device_id_type=pl.DeviceIdType.MESH,
        )
        rdma.start()
        rdma.wait()

        @pl.when(my > 0)
        def _():
            hcs = [comm_ref[bi] for bi in range(b)]
            for t8 in range(0, T_CORR, 8):
                cchunk = C_ref[:, t8:t8 + 8, :]
                for bi in range(b):
                    corrs = []
                    for tt in range(8):
                        hcs[bi] = hcs[bi] * dAv
                        ct = cchunk[bi, tt:tt + 1, :]
                        corrs.append(jax.lax.dot_general(
                            ct, hcs[bi], (((1,), (0,)), ((), ())),
                            preferred_element_type=jnp.float32
                        ).astype(cdt))
                    out_ref[bi, t8:t8 + 8, :] = (
                        out_ref[bi, t8:t8 + 8, :]
                        + jnp.concatenate(corrs, axis=0))

    return pl.pallas_call(
        body,
        out_shape=jax.ShapeDtypeStruct((b, s, d), cdt),
        in_specs=[
            pl.BlockSpec(memory_space=pltpu.VMEM),
            pl.BlockSpec(memory_space=pltpu.VMEM),
            pl.BlockSpec(memory_space=pltpu.VMEM),
            pl.BlockSpec(memory_space=pltpu.VMEM),
        ],
        out_specs=pl.BlockSpec(memory_space=pltpu.VMEM),
        scratch_shapes=[
            pltpu.VMEM((b, n, d), cdt),
            pltpu.VMEM((b, n, d), cdt),
            pltpu.SemaphoreType.DMA,
            pltpu.SemaphoreType.DMA,
        ],
        compiler_params=pltpu.CompilerParams(collective_id=0),
    )(x.astype(cdt), dA, Bt, Cr)


# device time: 183629 ns/iter; 3.3190x vs baseline; 3.3190x over previous
import jax
import jax.numpy as jnp
from jax import lax
from jax.experimental import pallas as pl
from jax.experimental.pallas import tpu as pltpu

N_DEV = 4
T_CORR = 64


def kernel(x, A, B, C):
    b, s, d = x.shape
    n = A.shape[1]

    cdt = jnp.bfloat16
    Bt = jnp.swapaxes(B, 1, 2).astype(cdt)
    Ct = jnp.swapaxes(C, 1, 2).astype(cdt)
    dA = jnp.exp(A.T).astype(cdt)

    def body(x_ref, dA_ref, B_ref, C_ref, out_ref,
             hout_ref, comm_ref, send_sem, recv_sem):
        my = lax.axis_index("i")
        left = (my - 1) % N_DEV
        right = (my + 1) % N_DEV

        barrier_sem = pltpu.get_barrier_semaphore()
        pl.semaphore_signal(barrier_sem, inc=1, device_id=(left,),
                            device_id_type=pl.DeviceIdType.MESH)
        pl.semaphore_signal(barrier_sem, inc=1, device_id=(right,),
                            device_id_type=pl.DeviceIdType.MESH)
        pl.semaphore_wait(barrier_sem, 2)

        dAv = dA_ref[:, :][None]

        BLK = 128
        NS = 2
        nh = n // NS

        for ni in range(NS):
            n0 = ni * nh
            dAh = dAv[:, n0:n0 + nh, :]

            def block_step(k, h, n0=n0, dAh=dAh, first=(ni == 0)):
                t0 = k * BLK
                bblk = B_ref[:, n0:n0 + nh, pl.ds(t0, BLK)]
                cblk = C_ref[:, n0:n0 + nh, pl.ds(t0, BLK)]
                for j8 in range(0, BLK, 8):
                    xchunk = x_ref[:, pl.ds(t0 + j8, 8), :]
                    ys = []
                    for jj in range(8):
                        j = j8 + jj
                        xt = xchunk[:, jj:jj + 1, :]
                        bt = bblk[:, :, j:j + 1]
                        ct = cblk[:, :, j:j + 1]
                        h = h * dAh + bt * xt
                        ys.append(jnp.sum(h * ct, axis=1, keepdims=True))
                    ynew = jnp.concatenate(ys, axis=1)
                    if first:
                        out_ref[:, pl.ds(t0 + j8, 8), :] = ynew
                    else:
                        out_ref[:, pl.ds(t0 + j8, 8), :] = (
                            out_ref[:, pl.ds(t0 + j8, 8), :] + ynew)
                return h

            h0 = jnp.zeros((b, nh, d), dtype=cdt)
            h_final = lax.fori_loop(0, s // BLK, block_step, h0)
            hout_ref[:, n0:n0 + nh, :] = h_final

        rdma = pltpu.make_async_remote_copy(
            src_ref=hout_ref,
            dst_ref=comm_ref,
            send_sem=send_sem,
            recv_sem=recv_sem,
            device_id=(right,),
            device_id_type=pl.DeviceIdType.MESH,
        )
        rdma.start()
        rdma.wait()

        @pl.when(my > 0)
        def _():
            cblk = C_ref[:, :, 0:T_CORR]
            hc = comm_ref[...]
            for t8 in range(0, T_CORR, 8):
                corrs = []
                for tt in range(8):
                    hc = hc * dAv
                    ct = cblk[:, :, t8 + tt:t8 + tt + 1]
                    corrs.append(jnp.sum(hc * ct, axis=1, keepdims=True))
                out_ref[:, t8:t8 + 8, :] = (
                    out_ref[:, t8:t8 + 8, :]
                    + jnp.concatenate(corrs, axis=1))

    return pl.pallas_call(
        body,
        out_shape=jax.ShapeDtypeStruct((b, s, d), cdt),
        in_specs=[
            pl.BlockSpec(memory_space=pltpu.VMEM),
            pl.BlockSpec(memory_space=pltpu.VMEM),
            pl.BlockSpec(memory_space=pltpu.VMEM),
            pl.BlockSpec(memory_space=pltpu.VMEM),
        ],
        out_specs=pl.BlockSpec(memory_space=pltpu.VMEM),
        scratch_shapes=[
            pltpu.VMEM((b, n, d), cdt),
            pltpu.VMEM((b, n, d), cdt),
            pltpu.SemaphoreType.DMA,
            pltpu.SemaphoreType.DMA,
        ],
        compiler_params=pltpu.CompilerParams(collective_id=0),
    )(x.astype(cdt), dA, Bt, Ct)
